# baseline (speedup 1.0000x reference)
import functools

import jax
import jax.numpy as jnp
from jax import lax
from jax.experimental import pallas as pl
from jax.experimental.pallas import tpu as pltpu
from jax.experimental.pallas import tpu_sc as plsc

_BATCH = 16384
_HIDDEN = 64
_NUM_CORES = 2
_NUM_SUBCORES = 16
_NW = _NUM_CORES * _NUM_SUBCORES
_BPW = _BATCH // _NW
_GRP = 16
_NGRP = _BPW // _GRP

_sc_mesh = plsc.VectorSubcoreMesh(core_axis_name="c", subcore_axis_name="s")


@functools.partial(
    pl.kernel,
    out_type=(
        jax.ShapeDtypeStruct((_BATCH, _HIDDEN), jnp.float32),
        jax.ShapeDtypeStruct((_BATCH, _HIDDEN), jnp.float32),
    ),
    mesh=_sc_mesh,
    scratch_types=[
        pltpu.VMEM((_BPW,), jnp.int32),
        pltpu.VMEM((_BPW,), jnp.int32),
        pltpu.SemaphoreType.DMA,
        pltpu.SemaphoreType.DMA,
    ],
)
def _sc_gather(uid_hbm, iid_hbm, utab_hbm, itab_hbm, uout_hbm, iout_hbm,
               uidx_v, iidx_v, usem, isem):
    wid = lax.axis_index("s") * _NUM_CORES + lax.axis_index("c")
    base = wid * _BPW
    pltpu.sync_copy(uid_hbm.at[pl.ds(base, _BPW)], uidx_v)
    pltpu.sync_copy(iid_hbm.at[pl.ds(base, _BPW)], iidx_v)

    def body(g, carry):
        uvec = uidx_v[pl.ds(g * _GRP, _GRP)]
        ivec = iidx_v[pl.ds(g * _GRP, _GRP)]
        dst0 = base + g * _GRP
        descs = []
        for j in range(_GRP):
            descs.append(pltpu.async_copy(
                utab_hbm.at[pl.ds(uvec[j], 1)],
                uout_hbm.at[pl.ds(dst0 + j, 1)], usem))
            descs.append(pltpu.async_copy(
                itab_hbm.at[pl.ds(ivec[j], 1)],
                iout_hbm.at[pl.ds(dst0 + j, 1)], isem))
        for d in descs:
            d.wait()
        return carry

    lax.fori_loop(0, _NGRP, body, 0)


_NRM_BLK = 2048


def _norm_body(uraw_ref, iraw_ref, uo_ref, io_ref):
    for raw_ref, o_ref in ((uraw_ref, uo_ref), (iraw_ref, io_ref)):
        x = raw_ref[...]
        nrm2 = jnp.sum(x * x, axis=1, keepdims=True)
        scale = jnp.where(nrm2 > 1e-24, lax.rsqrt(nrm2), 1e12)
        o_ref[...] = x * scale


_tc_normalize = pl.pallas_call(
    _norm_body,
    grid=(_BATCH // _NRM_BLK,),
    in_specs=[
        pl.BlockSpec((_NRM_BLK, _HIDDEN), lambda i: (i, 0)),
        pl.BlockSpec((_NRM_BLK, _HIDDEN), lambda i: (i, 0)),
    ],
    out_specs=[
        pl.BlockSpec((_NRM_BLK, _HIDDEN), lambda i: (i, 0)),
        pl.BlockSpec((_NRM_BLK, _HIDDEN), lambda i: (i, 0)),
    ],
    out_shape=(
        jax.ShapeDtypeStruct((_BATCH, _HIDDEN), jnp.float32),
        jax.ShapeDtypeStruct((_BATCH, _HIDDEN), jnp.float32),
    ),
)


def kernel(user_ids, item_ids, user_table, item_table):
    uid = user_ids.astype(jnp.int32)
    iid = item_ids.astype(jnp.int32)
    uraw, iraw = _sc_gather(uid, iid, user_table, item_table)
    return _tc_normalize(uraw, iraw)


# speedup vs baseline: 2.0911x; 2.0911x over previous
import functools

import jax
import jax.numpy as jnp
from jax import lax
from jax.experimental import pallas as pl
from jax.experimental.pallas import tpu as pltpu
from jax.experimental.pallas import tpu_sc as plsc

_BATCH = 16384
_HIDDEN = 64
_ROWS = 100001
_SPLIT = 50176
_RPK_BLK = 512
_RPK_STEPS = _SPLIT // _RPK_BLK
_NUM_CORES = 2
_NUM_SUBCORES = 16
_NW = _NUM_CORES * _NUM_SUBCORES
_BPW = _BATCH // _NW
_CHK = _BPW // 2


def _repack_body(u_ref, i_ref, uo_ref, io_ref):
    h = pl.program_id(1)

    @pl.when(h == 0)
    def _left():
        uo_ref[:, :_HIDDEN] = u_ref[...]
        io_ref[:, :_HIDDEN] = i_ref[...]

    @pl.when(h == 1)
    def _right():
        uo_ref[:, _HIDDEN:] = u_ref[...]
        io_ref[:, _HIDDEN:] = i_ref[...]


_tc_repack = pl.pallas_call(
    _repack_body,
    grid=(_RPK_STEPS, 2),
    in_specs=[
        pl.BlockSpec((_RPK_BLK, _HIDDEN), lambda i, h: (h * _RPK_STEPS + i, 0)),
        pl.BlockSpec((_RPK_BLK, _HIDDEN), lambda i, h: (h * _RPK_STEPS + i, 0)),
    ],
    out_specs=[
        pl.BlockSpec((_RPK_BLK, 2 * _HIDDEN), lambda i, h: (i, 0)),
        pl.BlockSpec((_RPK_BLK, 2 * _HIDDEN), lambda i, h: (i, 0)),
    ],
    out_shape=(
        jax.ShapeDtypeStruct((_SPLIT, 2 * _HIDDEN), jnp.float32),
        jax.ShapeDtypeStruct((_SPLIT, 2 * _HIDDEN), jnp.float32),
    ),
)


_sc_mesh = plsc.VectorSubcoreMesh(core_axis_name="c", subcore_axis_name="s")


@functools.partial(
    pl.kernel,
    out_type=(
        jax.ShapeDtypeStruct((_BATCH, 2 * _HIDDEN), jnp.float32),
        jax.ShapeDtypeStruct((_BATCH, 2 * _HIDDEN), jnp.float32),
    ),
    mesh=_sc_mesh,
    scratch_types=[
        pltpu.VMEM((_BPW,), jnp.int32),
        pltpu.VMEM((_BPW,), jnp.int32),
        pltpu.VMEM((_CHK, 2 * _HIDDEN), jnp.float32),
        pltpu.VMEM((_CHK, 2 * _HIDDEN), jnp.float32),
        pltpu.SemaphoreType.DMA,
        pltpu.SemaphoreType.DMA,
    ],
)
def _sc_gather(uj_hbm, ij_hbm, utab_hbm, itab_hbm, uout_hbm, iout_hbm,
               uidx_v, iidx_v, urows_v, irows_v, usem, isem):
    wid = lax.axis_index("s") * _NUM_CORES + lax.axis_index("c")
    base = wid * _BPW
    pltpu.sync_copy(uj_hbm.at[pl.ds(base, _BPW)], uidx_v)
    pltpu.sync_copy(ij_hbm.at[pl.ds(base, _BPW)], iidx_v)
    for c in range(2):
        off = c * _CHK
        ucp = pltpu.async_copy(utab_hbm.at[uidx_v.at[pl.ds(off, _CHK)]],
                               urows_v, usem)
        icp = pltpu.async_copy(itab_hbm.at[iidx_v.at[pl.ds(off, _CHK)]],
                               irows_v, isem)
        ucp.wait()
        pltpu.sync_copy(urows_v, uout_hbm.at[pl.ds(base + off, _CHK)])
        icp.wait()
        pltpu.sync_copy(irows_v, iout_hbm.at[pl.ds(base + off, _CHK)])


_NRM_BLK = 2048


def _norm_body(uraw_ref, iraw_ref, uk_ref, ik_ref, uo_ref, io_ref):
    for raw_ref, k_ref, o_ref in ((uraw_ref, uk_ref, uo_ref),
                                  (iraw_ref, ik_ref, io_ref)):
        raw = raw_ref[...]
        sel = jnp.where(k_ref[...] > 0.5, raw[:, _HIDDEN:], raw[:, :_HIDDEN])
        nrm2 = jnp.sum(sel * sel, axis=1, keepdims=True)
        scale = jnp.where(nrm2 > 1e-24, lax.rsqrt(nrm2), 1e12)
        o_ref[...] = sel * scale


_tc_normalize = pl.pallas_call(
    _norm_body,
    grid=(_BATCH // _NRM_BLK,),
    in_specs=[
        pl.BlockSpec((_NRM_BLK, 2 * _HIDDEN), lambda i: (i, 0)),
        pl.BlockSpec((_NRM_BLK, 2 * _HIDDEN), lambda i: (i, 0)),
        pl.BlockSpec((_NRM_BLK, 1), lambda i: (i, 0)),
        pl.BlockSpec((_NRM_BLK, 1), lambda i: (i, 0)),
    ],
    out_specs=[
        pl.BlockSpec((_NRM_BLK, _HIDDEN), lambda i: (i, 0)),
        pl.BlockSpec((_NRM_BLK, _HIDDEN), lambda i: (i, 0)),
    ],
    out_shape=(
        jax.ShapeDtypeStruct((_BATCH, _HIDDEN), jnp.float32),
        jax.ShapeDtypeStruct((_BATCH, _HIDDEN), jnp.float32),
    ),
)


def kernel(user_ids, item_ids, user_table, item_table):
    uid = user_ids.astype(jnp.int32)
    iid = item_ids.astype(jnp.int32)
    uj = jnp.where(uid < _SPLIT, uid, uid - _SPLIT)
    ij = jnp.where(iid < _SPLIT, iid, iid - _SPLIT)
    uk = (uid >= _SPLIT).astype(jnp.float32).reshape(_BATCH, 1)
    ik = (iid >= _SPLIT).astype(jnp.float32).reshape(_BATCH, 1)
    utab, itab = _tc_repack(user_table, item_table)
    uraw, iraw = _sc_gather(uj, ij, utab, itab)
    return _tc_normalize(uraw, iraw, uk, ik)


# speedup vs baseline: 4.6933x; 2.2444x over previous
import functools

import jax
import jax.numpy as jnp
from jax import lax
from jax.experimental import pallas as pl
from jax.experimental.pallas import tpu as pltpu
from jax.experimental.pallas import tpu_sc as plsc

_BATCH = 16384
_HIDDEN = 64
_ROWS = 100001
_SPLIT = 50176
_TPK_BLK = 512
_TPK_STEPS = _SPLIT // _TPK_BLK
_NUM_CORES = 2
_NUM_SUBCORES = 16
_NW = _NUM_CORES * _NUM_SUBCORES
_BPW = _BATCH // _NW
_CHK = _BPW // 2


def _eye(n):
    r = lax.broadcasted_iota(jnp.int32, (n, n), 0)
    c = lax.broadcasted_iota(jnp.int32, (n, n), 1)
    return (r == c).astype(jnp.float32)


def _tpk_body(ua_ref, ub_ref, ia_ref, ib_ref, uo_ref, io_ref):
    eye = _eye(_HIDDEN)
    dn = (((0,), (0,)), ((), ()))
    for a_ref, b_ref, o_ref in ((ua_ref, ub_ref, uo_ref),
                                (ia_ref, ib_ref, io_ref)):
        at = lax.dot_general(a_ref[...], eye, dn,
                             preferred_element_type=jnp.float32)
        bt = lax.dot_general(b_ref[...], eye, dn,
                             preferred_element_type=jnp.float32)
        o_ref[:, :_HIDDEN] = at
        o_ref[:, _HIDDEN:] = bt


_tc_transpack = pl.pallas_call(
    _tpk_body,
    grid=(_TPK_STEPS,),
    in_specs=[
        pl.BlockSpec((_HIDDEN, _TPK_BLK), lambda i: (0, i)),
        pl.BlockSpec((_HIDDEN, _TPK_BLK), lambda i: (0, _TPK_STEPS + i)),
        pl.BlockSpec((_HIDDEN, _TPK_BLK), lambda i: (0, i)),
        pl.BlockSpec((_HIDDEN, _TPK_BLK), lambda i: (0, _TPK_STEPS + i)),
    ],
    out_specs=[
        pl.BlockSpec((_TPK_BLK, 2 * _HIDDEN), lambda i: (i, 0)),
        pl.BlockSpec((_TPK_BLK, 2 * _HIDDEN), lambda i: (i, 0)),
    ],
    out_shape=(
        jax.ShapeDtypeStruct((_SPLIT, 2 * _HIDDEN), jnp.float32),
        jax.ShapeDtypeStruct((_SPLIT, 2 * _HIDDEN), jnp.float32),
    ),
)


_sc_mesh = plsc.VectorSubcoreMesh(core_axis_name="c", subcore_axis_name="s")


@functools.partial(
    pl.kernel,
    out_type=(
        jax.ShapeDtypeStruct((_BATCH, 2 * _HIDDEN), jnp.float32),
        jax.ShapeDtypeStruct((_BATCH, 2 * _HIDDEN), jnp.float32),
    ),
    mesh=_sc_mesh,
    scratch_types=[
        pltpu.VMEM((_BPW,), jnp.int32),
        pltpu.VMEM((_BPW,), jnp.int32),
        pltpu.VMEM((_CHK, 2 * _HIDDEN), jnp.float32),
        pltpu.VMEM((_CHK, 2 * _HIDDEN), jnp.float32),
        pltpu.SemaphoreType.DMA,
        pltpu.SemaphoreType.DMA,
    ],
)
def _sc_gather(uj_hbm, ij_hbm, utab_hbm, itab_hbm, uout_hbm, iout_hbm,
               uidx_v, iidx_v, urows_v, irows_v, usem, isem):
    wid = lax.axis_index("s") * _NUM_CORES + lax.axis_index("c")
    base = wid * _BPW
    pltpu.sync_copy(uj_hbm.at[pl.ds(base, _BPW)], uidx_v)
    pltpu.sync_copy(ij_hbm.at[pl.ds(base, _BPW)], iidx_v)
    for c in range(2):
        off = c * _CHK
        ucp = pltpu.async_copy(utab_hbm.at[uidx_v.at[pl.ds(off, _CHK)]],
                               urows_v, usem)
        icp = pltpu.async_copy(itab_hbm.at[iidx_v.at[pl.ds(off, _CHK)]],
                               irows_v, isem)
        ucp.wait()
        pltpu.sync_copy(urows_v, uout_hbm.at[pl.ds(base + off, _CHK)])
        icp.wait()
        pltpu.sync_copy(irows_v, iout_hbm.at[pl.ds(base + off, _CHK)])


_NRM_BLK = 2048


def _norm_body(uraw_ref, iraw_ref, uk_ref, ik_ref, uo_ref, io_ref):
    eye = _eye(2 * _HIDDEN)
    dn = (((1,), (1,)), ((), ()))
    for raw_ref, k_ref, o_ref in ((uraw_ref, uk_ref, uo_ref),
                                  (iraw_ref, ik_ref, io_ref)):
        raw_t = lax.dot_general(eye, raw_ref[...], dn,
                                preferred_element_type=jnp.float32)
        sel = jnp.where(k_ref[...] > 0.5, raw_t[_HIDDEN:, :], raw_t[:_HIDDEN, :])
        nrm2 = jnp.sum(sel * sel, axis=0, keepdims=True)
        scale = jnp.where(nrm2 > 1e-24, lax.rsqrt(nrm2), 1e12)
        o_ref[...] = sel * scale


_tc_normalize = pl.pallas_call(
    _norm_body,
    grid=(_BATCH // _NRM_BLK,),
    in_specs=[
        pl.BlockSpec((_NRM_BLK, 2 * _HIDDEN), lambda i: (i, 0)),
        pl.BlockSpec((_NRM_BLK, 2 * _HIDDEN), lambda i: (i, 0)),
        pl.BlockSpec((1, _NRM_BLK), lambda i: (0, i)),
        pl.BlockSpec((1, _NRM_BLK), lambda i: (0, i)),
    ],
    out_specs=[
        pl.BlockSpec((_HIDDEN, _NRM_BLK), lambda i: (0, i)),
        pl.BlockSpec((_HIDDEN, _NRM_BLK), lambda i: (0, i)),
    ],
    out_shape=(
        jax.ShapeDtypeStruct((_HIDDEN, _BATCH), jnp.float32),
        jax.ShapeDtypeStruct((_HIDDEN, _BATCH), jnp.float32),
    ),
)


def kernel(user_ids, item_ids, user_table, item_table):
    uid = user_ids.astype(jnp.int32)
    iid = item_ids.astype(jnp.int32)
    uj = jnp.where(uid < _SPLIT, uid, uid - _SPLIT)
    ij = jnp.where(iid < _SPLIT, iid, iid - _SPLIT)
    uk = (uid >= _SPLIT).astype(jnp.float32).reshape(1, _BATCH)
    ik = (iid >= _SPLIT).astype(jnp.float32).reshape(1, _BATCH)
    ut = user_table.T
    it = item_table.T
    utab, itab = _tc_transpack(ut, ut, it, it)
    uraw, iraw = _sc_gather(uj, ij, utab, itab)
    uo_t, io_t = _tc_normalize(uraw, iraw, uk, ik)
    return (uo_t.T, io_t.T)


# speedup vs baseline: 5.0275x; 1.0712x over previous
import functools

import jax
import jax.numpy as jnp
from jax import lax
from jax.experimental import pallas as pl
from jax.experimental.pallas import tpu as pltpu
from jax.experimental.pallas import tpu_sc as plsc

_BATCH = 16384
_HIDDEN = 64
_ROWS = 100001
_SPLIT = 50176
_TPK_BLK = 1024
_TPK_STEPS = _SPLIT // _TPK_BLK
_NUM_CORES = 2
_NUM_SUBCORES = 16
_NW = _NUM_CORES * _NUM_SUBCORES
_BPW = _BATCH // _NW
_CHK = _BPW // 2


def _eye(n):
    r = lax.broadcasted_iota(jnp.int32, (n, n), 0)
    c = lax.broadcasted_iota(jnp.int32, (n, n), 1)
    return (r == c).astype(jnp.float32)


def _tpk_body(a_ref, b_ref, o_ref):
    x = jnp.concatenate([a_ref[...], b_ref[...]], axis=0)
    o_ref[...] = lax.dot_general(x, _eye(2 * _HIDDEN), (((0,), (0,)), ((), ())),
                                 preferred_element_type=jnp.float32)


_tc_transpack = pl.pallas_call(
    _tpk_body,
    grid=(_TPK_STEPS,),
    in_specs=[
        pl.BlockSpec((_HIDDEN, _TPK_BLK), lambda i: (0, i)),
        pl.BlockSpec((_HIDDEN, _TPK_BLK), lambda i: (0, _TPK_STEPS + i)),
    ],
    out_specs=pl.BlockSpec((_TPK_BLK, 2 * _HIDDEN), lambda i: (i, 0)),
    out_shape=jax.ShapeDtypeStruct((_SPLIT, 2 * _HIDDEN), jnp.float32),
)


_sc_mesh = plsc.VectorSubcoreMesh(core_axis_name="c", subcore_axis_name="s")


@functools.partial(
    pl.kernel,
    out_type=jax.ShapeDtypeStruct((_BATCH, 2 * _HIDDEN), jnp.float32),
    mesh=_sc_mesh,
    scratch_types=[
        pltpu.VMEM((_BPW,), jnp.int32),
        pltpu.VMEM((_CHK, 2 * _HIDDEN), jnp.float32),
        pltpu.VMEM((_CHK, 2 * _HIDDEN), jnp.float32),
        pltpu.SemaphoreType.DMA,
        pltpu.SemaphoreType.DMA,
    ],
)
def _sc_gather(j_hbm, tab_hbm, out_hbm, idx_v, rows0_v, rows1_v, sem0, sem1):
    wid = lax.axis_index("s") * _NUM_CORES + lax.axis_index("c")
    base = wid * _BPW
    pltpu.sync_copy(j_hbm.at[pl.ds(base, _BPW)], idx_v)
    cp0 = pltpu.async_copy(tab_hbm.at[idx_v.at[pl.ds(0, _CHK)]], rows0_v, sem0)
    cp1 = pltpu.async_copy(tab_hbm.at[idx_v.at[pl.ds(_CHK, _CHK)]], rows1_v, sem1)
    cp0.wait()
    pltpu.sync_copy(rows0_v, out_hbm.at[pl.ds(base, _CHK)])
    cp1.wait()
    pltpu.sync_copy(rows1_v, out_hbm.at[pl.ds(base + _CHK, _CHK)])


_NRM_BLK = 2048


def _norm_body(raw_ref, k_ref, o_ref):
    raw_t = lax.dot_general(_eye(2 * _HIDDEN), raw_ref[...],
                            (((1,), (1,)), ((), ())),
                            preferred_element_type=jnp.float32)
    sel = jnp.where(k_ref[...] > 0.5, raw_t[_HIDDEN:, :], raw_t[:_HIDDEN, :])
    nrm2 = jnp.sum(sel * sel, axis=0, keepdims=True)
    scale = jnp.where(nrm2 > 1e-24, lax.rsqrt(nrm2), 1e12)
    o_ref[...] = sel * scale


_tc_normalize = pl.pallas_call(
    _norm_body,
    grid=(_BATCH // _NRM_BLK,),
    in_specs=[
        pl.BlockSpec((_NRM_BLK, 2 * _HIDDEN), lambda i: (i, 0)),
        pl.BlockSpec((1, _NRM_BLK), lambda i: (0, i)),
    ],
    out_specs=pl.BlockSpec((_HIDDEN, _NRM_BLK), lambda i: (0, i)),
    out_shape=jax.ShapeDtypeStruct((_HIDDEN, _BATCH), jnp.float32),
)


def kernel(user_ids, item_ids, user_table, item_table):
    uid = user_ids.astype(jnp.int32)
    iid = item_ids.astype(jnp.int32)
    uj = jnp.where(uid < _SPLIT, uid, uid - _SPLIT)
    ij = jnp.where(iid < _SPLIT, iid, iid - _SPLIT)
    uk = (uid >= _SPLIT).astype(jnp.float32).reshape(1, _BATCH)
    ik = (iid >= _SPLIT).astype(jnp.float32).reshape(1, _BATCH)
    ut = user_table.T
    it = item_table.T
    utab = _tc_transpack(ut, ut)
    uraw = _sc_gather(uj, utab)
    itab = _tc_transpack(it, it)
    iraw = _sc_gather(ij, itab)
    uo_t = _tc_normalize(uraw, uk)
    io_t = _tc_normalize(iraw, ik)
    return (uo_t.T, io_t.T)


# speedup vs baseline: 6.2463x; 1.2424x over previous
import functools

import jax
import jax.numpy as jnp
from jax import lax
from jax.experimental import pallas as pl
from jax.experimental.pallas import tpu as pltpu
from jax.experimental.pallas import tpu_sc as plsc

_BATCH = 16384
_HIDDEN = 64
_ROWS = 100001
_SPLIT = 51200
_TPK_BLK = 2048
_TPK_STEPS = _SPLIT // _TPK_BLK
_NUM_CORES = 2
_NUM_SUBCORES = 16
_NW = _NUM_CORES * _NUM_SUBCORES
_BPW = _BATCH // _NW
_CHK = _BPW // 2


def _eye(n):
    r = lax.broadcasted_iota(jnp.int32, (n, n), 0)
    c = lax.broadcasted_iota(jnp.int32, (n, n), 1)
    return (r == c).astype(jnp.float32)


def _tpk_body(a_ref, b_ref, o_ref):
    x = jnp.concatenate([a_ref[...], b_ref[...]], axis=0)
    o_ref[...] = lax.dot_general(x, _eye(2 * _HIDDEN), (((0,), (0,)), ((), ())),
                                 preferred_element_type=jnp.float32)


_tc_transpack = pl.pallas_call(
    _tpk_body,
    grid=(_TPK_STEPS,),
    in_specs=[
        pl.BlockSpec((_HIDDEN, _TPK_BLK), lambda i: (0, i)),
        pl.BlockSpec((_HIDDEN, _TPK_BLK),
                     lambda i: (0, jnp.minimum(_TPK_STEPS + i, 48))),
    ],
    out_specs=pl.BlockSpec((_TPK_BLK, 2 * _HIDDEN), lambda i: (i, 0)),
    out_shape=jax.ShapeDtypeStruct((_SPLIT, 2 * _HIDDEN), jnp.float32),
)


_sc_mesh = plsc.VectorSubcoreMesh(core_axis_name="c", subcore_axis_name="s")


@functools.partial(
    pl.kernel,
    out_type=jax.ShapeDtypeStruct((_BATCH, 2 * _HIDDEN), jnp.float32),
    mesh=_sc_mesh,
    scratch_types=[
        pltpu.VMEM((_BPW,), jnp.int32),
        pltpu.VMEM((_CHK, 2 * _HIDDEN), jnp.float32),
        pltpu.VMEM((_CHK, 2 * _HIDDEN), jnp.float32),
        pltpu.SemaphoreType.DMA,
        pltpu.SemaphoreType.DMA,
    ],
)
def _sc_gather(j_hbm, tab_hbm, out_hbm, idx_v, rows0_v, rows1_v, sem0, sem1):
    wid = lax.axis_index("s") * _NUM_CORES + lax.axis_index("c")
    base = wid * _BPW
    pltpu.sync_copy(j_hbm.at[pl.ds(base, _BPW)], idx_v)
    cp0 = pltpu.async_copy(tab_hbm.at[idx_v.at[pl.ds(0, _CHK)]], rows0_v, sem0)
    cp1 = pltpu.async_copy(tab_hbm.at[idx_v.at[pl.ds(_CHK, _CHK)]], rows1_v, sem1)
    cp0.wait()
    pltpu.sync_copy(rows0_v, out_hbm.at[pl.ds(base, _CHK)])
    cp1.wait()
    pltpu.sync_copy(rows1_v, out_hbm.at[pl.ds(base + _CHK, _CHK)])


_NRM_BLK = 2048


def _norm_body(raw_ref, k_ref, o_ref):
    raw_t = lax.dot_general(_eye(2 * _HIDDEN), raw_ref[...],
                            (((1,), (1,)), ((), ())),
                            preferred_element_type=jnp.float32)
    sel = jnp.where(k_ref[...] > 0.5, raw_t[_HIDDEN:, :], raw_t[:_HIDDEN, :])
    nrm2 = jnp.sum(sel * sel, axis=0, keepdims=True)
    scale = jnp.where(nrm2 > 1e-24, lax.rsqrt(nrm2), 1e12)
    o_ref[...] = sel * scale


_tc_normalize = pl.pallas_call(
    _norm_body,
    grid=(_BATCH // _NRM_BLK,),
    in_specs=[
        pl.BlockSpec((_NRM_BLK, 2 * _HIDDEN), lambda i: (i, 0)),
        pl.BlockSpec((1, _NRM_BLK), lambda i: (0, i)),
    ],
    out_specs=pl.BlockSpec((_HIDDEN, _NRM_BLK), lambda i: (0, i)),
    out_shape=jax.ShapeDtypeStruct((_HIDDEN, _BATCH), jnp.float32),
)


def kernel(user_ids, item_ids, user_table, item_table):
    uid = user_ids.astype(jnp.int32)
    iid = item_ids.astype(jnp.int32)
    uj = jnp.where(uid < _SPLIT, uid, uid - _SPLIT)
    ij = jnp.where(iid < _SPLIT, iid, iid - _SPLIT)
    uk = (uid >= _SPLIT).astype(jnp.float32).reshape(1, _BATCH)
    ik = (iid >= _SPLIT).astype(jnp.float32).reshape(1, _BATCH)
    ut = user_table.T
    it = item_table.T
    utab = _tc_transpack(ut, ut)
    uraw = _sc_gather(uj, utab)
    itab = _tc_transpack(it, it)
    iraw = _sc_gather(ij, itab)
    uo_t = _tc_normalize(uraw, uk)
    io_t = _tc_normalize(iraw, ik)
    return (uo_t.T, io_t.T)


# speedup vs baseline: 7.2639x; 1.1629x over previous
import functools

import jax
import jax.numpy as jnp
from jax import lax
from jax.experimental import pallas as pl
from jax.experimental.pallas import tpu as pltpu
from jax.experimental.pallas import tpu_sc as plsc

_BATCH = 16384
_HIDDEN = 64
_ROWS = 100001
_SPLIT = 53248
_TPK_BLK = 4096
_TPK_STEPS = _SPLIT // _TPK_BLK
_NUM_CORES = 2
_NUM_SUBCORES = 16
_NW = _NUM_CORES * _NUM_SUBCORES
_BPW = _BATCH // _NW
_CHK = _BPW // 2


def _eye(n):
    r = lax.broadcasted_iota(jnp.int32, (n, n), 0)
    c = lax.broadcasted_iota(jnp.int32, (n, n), 1)
    return (r == c).astype(jnp.float32)


def _tpk_body(a_ref, b_ref, o_ref):
    x = jnp.concatenate([a_ref[...], b_ref[...]], axis=0)
    o_ref[...] = lax.dot_general(x, _eye(2 * _HIDDEN), (((0,), (0,)), ((), ())),
                                 preferred_element_type=jnp.float32)


_tc_transpack = pl.pallas_call(
    _tpk_body,
    grid=(_TPK_STEPS,),
    in_specs=[
        pl.BlockSpec((_HIDDEN, _TPK_BLK), lambda i: (0, i)),
        pl.BlockSpec((_HIDDEN, _TPK_BLK),
                     lambda i: (0, jnp.minimum(_TPK_STEPS + i, 24))),
    ],
    out_specs=pl.BlockSpec((_TPK_BLK, 2 * _HIDDEN), lambda i: (i, 0)),
    out_shape=jax.ShapeDtypeStruct((_SPLIT, 2 * _HIDDEN), jnp.float32),
)


_sc_mesh = plsc.VectorSubcoreMesh(core_axis_name="c", subcore_axis_name="s")


@functools.partial(
    pl.kernel,
    out_type=jax.ShapeDtypeStruct((_BATCH, 2 * _HIDDEN), jnp.float32),
    mesh=_sc_mesh,
    scratch_types=[
        pltpu.VMEM((_BPW,), jnp.int32),
        pltpu.VMEM((_CHK, 2 * _HIDDEN), jnp.float32),
        pltpu.VMEM((_CHK, 2 * _HIDDEN), jnp.float32),
        pltpu.SemaphoreType.DMA,
        pltpu.SemaphoreType.DMA,
    ],
)
def _sc_gather(j_hbm, tab_hbm, out_hbm, idx_v, rows0_v, rows1_v, sem0, sem1):
    wid = lax.axis_index("s") * _NUM_CORES + lax.axis_index("c")
    base = wid * _BPW
    pltpu.sync_copy(j_hbm.at[pl.ds(base, _BPW)], idx_v)
    cp0 = pltpu.async_copy(tab_hbm.at[idx_v.at[pl.ds(0, _CHK)]], rows0_v, sem0)
    cp1 = pltpu.async_copy(tab_hbm.at[idx_v.at[pl.ds(_CHK, _CHK)]], rows1_v, sem1)
    cp0.wait()
    pltpu.sync_copy(rows0_v, out_hbm.at[pl.ds(base, _CHK)])
    cp1.wait()
    pltpu.sync_copy(rows1_v, out_hbm.at[pl.ds(base + _CHK, _CHK)])


_NRM_BLK = 2048


def _norm_body(raw_ref, k_ref, o_ref):
    raw_t = lax.dot_general(_eye(2 * _HIDDEN), raw_ref[...],
                            (((1,), (1,)), ((), ())),
                            preferred_element_type=jnp.float32)
    sel = jnp.where(k_ref[...] > 0.5, raw_t[_HIDDEN:, :], raw_t[:_HIDDEN, :])
    nrm2 = jnp.sum(sel * sel, axis=0, keepdims=True)
    scale = jnp.where(nrm2 > 1e-24, lax.rsqrt(nrm2), 1e12)
    o_ref[...] = sel * scale


_tc_normalize = pl.pallas_call(
    _norm_body,
    grid=(_BATCH // _NRM_BLK,),
    in_specs=[
        pl.BlockSpec((_NRM_BLK, 2 * _HIDDEN), lambda i: (i, 0)),
        pl.BlockSpec((1, _NRM_BLK), lambda i: (0, i)),
    ],
    out_specs=pl.BlockSpec((_HIDDEN, _NRM_BLK), lambda i: (0, i)),
    out_shape=jax.ShapeDtypeStruct((_HIDDEN, _BATCH), jnp.float32),
)


def kernel(user_ids, item_ids, user_table, item_table):
    uid = user_ids.astype(jnp.int32)
    iid = item_ids.astype(jnp.int32)
    uj = jnp.where(uid < _SPLIT, uid, uid - _SPLIT)
    ij = jnp.where(iid < _SPLIT, iid, iid - _SPLIT)
    uk = (uid >= _SPLIT).astype(jnp.float32).reshape(1, _BATCH)
    ik = (iid >= _SPLIT).astype(jnp.float32).reshape(1, _BATCH)
    ut = user_table.T
    it = item_table.T
    utab = _tc_transpack(ut, ut)
    uraw = _sc_gather(uj, utab)
    itab = _tc_transpack(it, it)
    iraw = _sc_gather(ij, itab)
    uo_t = _tc_normalize(uraw, uk)
    io_t = _tc_normalize(iraw, ik)
    return (uo_t.T, io_t.T)


# speedup vs baseline: 7.5814x; 1.0437x over previous
import functools

import jax
import jax.numpy as jnp
from jax import lax
from jax.experimental import pallas as pl
from jax.experimental.pallas import tpu as pltpu
from jax.experimental.pallas import tpu_sc as plsc

_BATCH = 16384
_HIDDEN = 64
_ROWS = 100001
_SPLIT = 57344
_TPK_BLK = 8192
_TPK_STEPS = _SPLIT // _TPK_BLK
_NUM_CORES = 2
_NUM_SUBCORES = 16
_NW = _NUM_CORES * _NUM_SUBCORES
_BPW = _BATCH // _NW
_CHK = _BPW // 2


def _eye(n):
    r = lax.broadcasted_iota(jnp.int32, (n, n), 0)
    c = lax.broadcasted_iota(jnp.int32, (n, n), 1)
    return (r == c).astype(jnp.float32)


def _tpk_body(a_ref, b_ref, o_ref):
    x = jnp.concatenate([a_ref[...], b_ref[...]], axis=0)
    o_ref[...] = lax.dot_general(x, _eye(2 * _HIDDEN), (((0,), (0,)), ((), ())),
                                 preferred_element_type=jnp.float32)


_tc_transpack = pl.pallas_call(
    _tpk_body,
    grid=(_TPK_STEPS,),
    in_specs=[
        pl.BlockSpec((_HIDDEN, _TPK_BLK), lambda i: (0, i)),
        pl.BlockSpec((_HIDDEN, _TPK_BLK),
                     lambda i: (0, jnp.minimum(_TPK_STEPS + i, 12))),
    ],
    out_specs=pl.BlockSpec((_TPK_BLK, 2 * _HIDDEN), lambda i: (i, 0)),
    out_shape=jax.ShapeDtypeStruct((_SPLIT, 2 * _HIDDEN), jnp.float32),
)


_sc_mesh = plsc.VectorSubcoreMesh(core_axis_name="c", subcore_axis_name="s")


@functools.partial(
    pl.kernel,
    out_type=jax.ShapeDtypeStruct((_BATCH, 2 * _HIDDEN), jnp.float32),
    mesh=_sc_mesh,
    scratch_types=[
        pltpu.VMEM((_BPW,), jnp.int32),
        pltpu.VMEM((_CHK, 2 * _HIDDEN), jnp.float32),
        pltpu.VMEM((_CHK, 2 * _HIDDEN), jnp.float32),
        pltpu.SemaphoreType.DMA,
        pltpu.SemaphoreType.DMA,
    ],
)
def _sc_gather(j_hbm, tab_hbm, out_hbm, idx_v, rows0_v, rows1_v, sem0, sem1):
    wid = lax.axis_index("s") * _NUM_CORES + lax.axis_index("c")
    base = wid * _BPW
    pltpu.sync_copy(j_hbm.at[pl.ds(base, _BPW)], idx_v)
    cp0 = pltpu.async_copy(tab_hbm.at[idx_v.at[pl.ds(0, _CHK)]], rows0_v, sem0)
    cp1 = pltpu.async_copy(tab_hbm.at[idx_v.at[pl.ds(_CHK, _CHK)]], rows1_v, sem1)
    cp0.wait()
    pltpu.sync_copy(rows0_v, out_hbm.at[pl.ds(base, _CHK)])
    cp1.wait()
    pltpu.sync_copy(rows1_v, out_hbm.at[pl.ds(base + _CHK, _CHK)])


_NRM_BLK = 2048


def _norm_body(raw_ref, k_ref, o_ref):
    raw_t = lax.dot_general(_eye(2 * _HIDDEN), raw_ref[...],
                            (((1,), (1,)), ((), ())),
                            preferred_element_type=jnp.float32)
    sel = jnp.where(k_ref[...] > 0.5, raw_t[_HIDDEN:, :], raw_t[:_HIDDEN, :])
    nrm2 = jnp.sum(sel * sel, axis=0, keepdims=True)
    scale = jnp.where(nrm2 > 1e-24, lax.rsqrt(nrm2), 1e12)
    o_ref[...] = sel * scale


_tc_normalize = pl.pallas_call(
    _norm_body,
    grid=(_BATCH // _NRM_BLK,),
    in_specs=[
        pl.BlockSpec((_NRM_BLK, 2 * _HIDDEN), lambda i: (i, 0)),
        pl.BlockSpec((1, _NRM_BLK), lambda i: (0, i)),
    ],
    out_specs=pl.BlockSpec((_HIDDEN, _NRM_BLK), lambda i: (0, i)),
    out_shape=jax.ShapeDtypeStruct((_HIDDEN, _BATCH), jnp.float32),
)


def kernel(user_ids, item_ids, user_table, item_table):
    uid = user_ids.astype(jnp.int32)
    iid = item_ids.astype(jnp.int32)
    uj = jnp.where(uid < _SPLIT, uid, uid - _SPLIT)
    ij = jnp.where(iid < _SPLIT, iid, iid - _SPLIT)
    uk = (uid >= _SPLIT).astype(jnp.float32).reshape(1, _BATCH)
    ik = (iid >= _SPLIT).astype(jnp.float32).reshape(1, _BATCH)
    ut = user_table.T
    it = item_table.T
    utab = _tc_transpack(ut, ut)
    uraw = _sc_gather(uj, utab)
    itab = _tc_transpack(it, it)
    iraw = _sc_gather(ij, itab)
    uo_t = _tc_normalize(uraw, uk)
    io_t = _tc_normalize(iraw, ik)
    return (uo_t.T, io_t.T)


# speedup vs baseline: 7.7056x; 1.0164x over previous
import functools

import jax
import jax.numpy as jnp
from jax import lax
from jax.experimental import pallas as pl
from jax.experimental.pallas import tpu as pltpu
from jax.experimental.pallas import tpu_sc as plsc

_BATCH = 16384
_HIDDEN = 64
_ROWS = 100001
_SPLIT = 50176
_TPK_BLK = 7168
_TPK_STEPS = _SPLIT // _TPK_BLK
_NUM_CORES = 2
_NUM_SUBCORES = 16
_NW = _NUM_CORES * _NUM_SUBCORES
_BPW = _BATCH // _NW
_CHK = _BPW // 2


def _eye(n):
    r = lax.broadcasted_iota(jnp.int32, (n, n), 0)
    c = lax.broadcasted_iota(jnp.int32, (n, n), 1)
    return (r == c).astype(jnp.float32)


def _tpk_body(a_ref, b_ref, o_ref):
    x = jnp.concatenate([a_ref[...], b_ref[...]], axis=0)
    o_ref[...] = lax.dot_general(x, _eye(2 * _HIDDEN), (((0,), (0,)), ((), ())),
                                 preferred_element_type=jnp.float32)


_tc_transpack = pl.pallas_call(
    _tpk_body,
    grid=(_TPK_STEPS,),
    in_specs=[
        pl.BlockSpec((_HIDDEN, _TPK_BLK), lambda i: (0, i)),
        pl.BlockSpec((_HIDDEN, _TPK_BLK), lambda i: (0, _TPK_STEPS + i)),
    ],
    out_specs=pl.BlockSpec((_TPK_BLK, 2 * _HIDDEN), lambda i: (i, 0)),
    out_shape=jax.ShapeDtypeStruct((_SPLIT, 2 * _HIDDEN), jnp.float32),
)


_sc_mesh = plsc.VectorSubcoreMesh(core_axis_name="c", subcore_axis_name="s")


@functools.partial(
    pl.kernel,
    out_type=jax.ShapeDtypeStruct((_BATCH, 2 * _HIDDEN), jnp.float32),
    mesh=_sc_mesh,
    scratch_types=[
        pltpu.VMEM((_BPW,), jnp.int32),
        pltpu.VMEM((_CHK, 2 * _HIDDEN), jnp.float32),
        pltpu.VMEM((_CHK, 2 * _HIDDEN), jnp.float32),
        pltpu.SemaphoreType.DMA,
        pltpu.SemaphoreType.DMA,
    ],
)
def _sc_gather(j_hbm, tab_hbm, out_hbm, idx_v, rows0_v, rows1_v, sem0, sem1):
    wid = lax.axis_index("s") * _NUM_CORES + lax.axis_index("c")
    base = wid * _BPW
    pltpu.sync_copy(j_hbm.at[pl.ds(base, _BPW)], idx_v)
    cp0 = pltpu.async_copy(tab_hbm.at[idx_v.at[pl.ds(0, _CHK)]], rows0_v, sem0)
    cp1 = pltpu.async_copy(tab_hbm.at[idx_v.at[pl.ds(_CHK, _CHK)]], rows1_v, sem1)
    cp0.wait()
    pltpu.sync_copy(rows0_v, out_hbm.at[pl.ds(base, _CHK)])
    cp1.wait()
    pltpu.sync_copy(rows1_v, out_hbm.at[pl.ds(base + _CHK, _CHK)])


_NRM_BLK = 2048


def _norm_body(raw_ref, k_ref, o_ref):
    raw_t = lax.dot_general(_eye(2 * _HIDDEN), raw_ref[...],
                            (((1,), (1,)), ((), ())),
                            preferred_element_type=jnp.float32)
    sel = jnp.where(k_ref[...] > 0.5, raw_t[_HIDDEN:, :], raw_t[:_HIDDEN, :])
    nrm2 = jnp.sum(sel * sel, axis=0, keepdims=True)
    scale = jnp.where(nrm2 > 1e-24, lax.rsqrt(nrm2), 1e12)
    o_ref[...] = sel * scale


_tc_normalize = pl.pallas_call(
    _norm_body,
    grid=(_BATCH // _NRM_BLK,),
    in_specs=[
        pl.BlockSpec((_NRM_BLK, 2 * _HIDDEN), lambda i: (i, 0)),
        pl.BlockSpec((1, _NRM_BLK), lambda i: (0, i)),
    ],
    out_specs=pl.BlockSpec((_HIDDEN, _NRM_BLK), lambda i: (0, i)),
    out_shape=jax.ShapeDtypeStruct((_HIDDEN, _BATCH), jnp.float32),
)


def kernel(user_ids, item_ids, user_table, item_table):
    uid = user_ids.astype(jnp.int32)
    iid = item_ids.astype(jnp.int32)
    uj = jnp.where(uid < _SPLIT, uid, uid - _SPLIT)
    ij = jnp.where(iid < _SPLIT, iid, iid - _SPLIT)
    uk = (uid >= _SPLIT).astype(jnp.float32).reshape(1, _BATCH)
    ik = (iid >= _SPLIT).astype(jnp.float32).reshape(1, _BATCH)
    ut = user_table.T
    it = item_table.T
    utab = _tc_transpack(ut, ut)
    uraw = _sc_gather(uj, utab)
    itab = _tc_transpack(it, it)
    iraw = _sc_gather(ij, itab)
    uo_t = _tc_normalize(uraw, uk)
    io_t = _tc_normalize(iraw, ik)
    return (uo_t.T, io_t.T)
